# baseline (speedup 1.0000x reference)
import functools

import jax
import jax.numpy as jnp
from jax import lax
from jax.experimental import pallas as pl
from jax.experimental.pallas import tpu as pltpu
from jax.experimental.pallas import tpu_sc as plsc

VOCAB = 100000
EMB = 128
BATCH = 1024

_NC, _NS = 2, 16
_NW = _NC * _NS
_B_PER_W = BATCH // _NW


def _sc_gather_body(table_hbm, idx_hbm, out_hbm, idx_v, rows_v, sem):
    wid = lax.axis_index("s") * _NC + lax.axis_index("c")
    base = wid * _B_PER_W
    pltpu.sync_copy(idx_hbm.at[pl.ds(base, _B_PER_W)], idx_v)
    pltpu.async_copy(table_hbm.at[idx_v], rows_v, sem).wait()
    pltpu.sync_copy(rows_v, out_hbm.at[pl.ds(base, _B_PER_W)])


@functools.cache
def _make_sc_gather():
    return pl.kernel(
        _sc_gather_body,
        out_type=jax.ShapeDtypeStruct((BATCH, EMB), jnp.float32),
        mesh=plsc.VectorSubcoreMesh(
            core_axis_name="c", subcore_axis_name="s",
            num_cores=_NC, num_subcores=_NS,
        ),
        scratch_types=[
            pltpu.VMEM((_B_PER_W,), jnp.int32),
            pltpu.VMEM((_B_PER_W, EMB), jnp.float32),
            pltpu.SemaphoreType.DMA,
        ],
    )


_TN = 2048
_NFULL = VOCAB // _TN
_EDGE = VOCAB - _NFULL * _TN
_N_TILES = _NFULL + 1
_NBUF = 4


def _matmul_body(emb_ref, w_hbm, b_hbm, out_hbm,
                 bufs, w_bufs, b_bufs, ebuf, w_ebuf, b_ebuf,
                 sems, wsems, esems):
    j = pl.program_id(0)
    slot = lax.rem(j, _NBUF)
    ws = lax.rem(j, 2)

    def w_copies(jj, s):
        cw = pltpu.make_async_copy(
            w_hbm.at[:, pl.ds(jj * _TN, _TN)], w_bufs.at[s], wsems.at[s])
        cb = pltpu.make_async_copy(
            b_hbm.at[:, pl.ds(jj * _TN, _TN)], b_bufs.at[s], wsems.at[2 + s])
        return cw, cb

    def edge_copies():
        cw = pltpu.make_async_copy(
            w_hbm.at[:, pl.ds(_NFULL * _TN, _EDGE)], w_ebuf, esems.at[0])
        cb = pltpu.make_async_copy(
            b_hbm.at[:, pl.ds(_NFULL * _TN, _EDGE)], b_ebuf, esems.at[1])
        return cw, cb

    @pl.when(j == 0)
    def _():
        cw, cb = w_copies(0, 0)
        cw.start()
        cb.start()

    @pl.when(j + 1 < _NFULL)
    def _():
        cw, cb = w_copies(j + 1, lax.rem(j + 1, 2))
        cw.start()
        cb.start()

    @pl.when(j + 1 == _NFULL)
    def _():
        cw, cb = edge_copies()
        cw.start()
        cb.start()

    @pl.when(j >= _NBUF)
    def _():
        pltpu.make_async_copy(
            bufs.at[lax.rem(j, _NBUF)],
            out_hbm.at[:, pl.ds((j - _NBUF) * _TN, _TN)],
            sems.at[lax.rem(j, _NBUF)],
        ).wait()

    @pl.when(j < _NFULL)
    def _():
        cw, cb = w_copies(j, ws)
        cw.wait()
        cb.wait()
        res = (
            jnp.dot(emb_ref[...], w_bufs[ws],
                    preferred_element_type=jnp.float32)
            + b_bufs[ws]
        )
        bufs[pl.ds(slot, 1)] = res[None]
        pltpu.make_async_copy(
            bufs.at[slot],
            out_hbm.at[:, pl.ds(j * _TN, _TN)],
            sems.at[slot],
        ).start()

    @pl.when(j == _NFULL)
    def _():
        cw, cb = edge_copies()
        cw.wait()
        cb.wait()
        res = (
            jnp.dot(emb_ref[...], w_ebuf[...],
                    preferred_element_type=jnp.float32)
            + b_ebuf[...]
        )
        ebuf[...] = res
        ec = pltpu.make_async_copy(
            ebuf, out_hbm.at[:, pl.ds(_NFULL * _TN, _EDGE)], esems.at[2])
        ec.start()
        for jj in range(_NFULL - _NBUF + 1, _NFULL):
            s = jj % _NBUF
            pltpu.make_async_copy(
                bufs.at[s],
                out_hbm.at[:, pl.ds(jj * _TN, _TN)],
                sems.at[s],
            ).wait()
        ec.wait()


_TM = 1024
_TNS = 4096
_GM = (BATCH + _TM - 1) // _TM
_GN = (VOCAB + _TNS - 1) // _TNS


def _mm_simple(emb_ref, w_ref, b_ref, out_ref):
    out_ref[...] = (
        jnp.dot(emb_ref[...], w_ref[...], preferred_element_type=jnp.float32)
        + b_ref[...]
    )


@jax.jit
def kernel(inputs, E, W, b):
    emb = _make_sc_gather()(E, inputs.astype(jnp.int32))
    b2 = b.reshape(1, VOCAB)
    return pl.pallas_call(
        _mm_simple,
        grid=(_GM, _GN),
        in_specs=[
            pl.BlockSpec((_TM, EMB), lambda i, j: (i, 0)),
            pl.BlockSpec((EMB, _TNS), lambda i, j: (0, j)),
            pl.BlockSpec((1, _TNS), lambda i, j: (0, j)),
        ],
        out_specs=pl.BlockSpec((_TM, _TNS), lambda i, j: (i, j)),
        out_shape=jax.ShapeDtypeStruct((BATCH, VOCAB), jnp.float32),
        compiler_params=pltpu.CompilerParams(
            dimension_semantics=("parallel", "parallel"),
        ),
    )(emb, W, b2)


@jax.jit
def _kernel_full(inputs, E, W, b):
    emb = _make_sc_gather()(E, inputs.astype(jnp.int32))
    b2 = b.reshape(1, VOCAB)
    logits = pl.pallas_call(
        _matmul_body,
        grid=(_N_TILES,),
        in_specs=[
            pl.BlockSpec((BATCH, EMB), lambda j: (0, 0)),
            pl.BlockSpec(memory_space=pl.ANY),
            pl.BlockSpec(memory_space=pl.ANY),
        ],
        out_specs=pl.BlockSpec(memory_space=pl.ANY),
        out_shape=jax.ShapeDtypeStruct((BATCH, VOCAB), jnp.float32),
        scratch_shapes=[
            pltpu.VMEM((_NBUF, BATCH, _TN), jnp.float32),
            pltpu.VMEM((2, EMB, _TN), jnp.float32),
            pltpu.VMEM((2, 1, _TN), jnp.float32),
            pltpu.VMEM((BATCH, _EDGE), jnp.float32),
            pltpu.VMEM((EMB, _EDGE), jnp.float32),
            pltpu.VMEM((1, _EDGE), jnp.float32),
            pltpu.SemaphoreType.DMA((_NBUF,)),
            pltpu.SemaphoreType.DMA((4,)),
            pltpu.SemaphoreType.DMA((3,)),
        ],
        compiler_params=pltpu.CompilerParams(
            dimension_semantics=("arbitrary",),
        ),
    )(emb, W, b2)
    return logits


# speedup vs baseline: 2.5268x; 2.5268x over previous
import functools

import jax
import jax.numpy as jnp
from jax import lax
from jax.experimental import pallas as pl
from jax.experimental.pallas import tpu as pltpu
from jax.experimental.pallas import tpu_sc as plsc

VOCAB = 100000
EMB = 128
BATCH = 1024

_NC, _NS = 2, 16
_NW = _NC * _NS
_B_PER_W = BATCH // _NW


def _sc_gather_body(table_hbm, idx_hbm, out_hbm, idx_v, rows_v, sem):
    wid = lax.axis_index("s") * _NC + lax.axis_index("c")
    base = wid * _B_PER_W
    pltpu.sync_copy(idx_hbm.at[pl.ds(base, _B_PER_W)], idx_v)
    pltpu.async_copy(table_hbm.at[idx_v], rows_v, sem).wait()
    pltpu.sync_copy(rows_v, out_hbm.at[pl.ds(base, _B_PER_W)])


@functools.cache
def _make_sc_gather():
    return pl.kernel(
        _sc_gather_body,
        out_type=jax.ShapeDtypeStruct((BATCH, EMB), jnp.float32),
        mesh=plsc.VectorSubcoreMesh(
            core_axis_name="c", subcore_axis_name="s",
            num_cores=_NC, num_subcores=_NS,
        ),
        scratch_types=[
            pltpu.VMEM((_B_PER_W,), jnp.int32),
            pltpu.VMEM((_B_PER_W, EMB), jnp.float32),
            pltpu.SemaphoreType.DMA,
        ],
    )


_TBN = 2048
_N_TILES = (VOCAB + _TBN - 1) // _TBN


def _mmT_body(w_ref, emb_ref, b_ref, out_ref):
    out_ref[...] = (
        lax.dot_general(
            w_ref[...], emb_ref[...],
            dimension_numbers=(((1,), (1,)), ((), ())),
            preferred_element_type=jnp.float32,
        )
        + b_ref[...]
    )


@jax.jit
def kernel(inputs, E, W, b):
    emb = _make_sc_gather()(E, inputs.astype(jnp.int32))
    w_t = W.T
    b2 = b.reshape(VOCAB, 1)
    out_t = pl.pallas_call(
        _mmT_body,
        grid=(_N_TILES,),
        in_specs=[
            pl.BlockSpec((_TBN, EMB), lambda j: (j, 0)),
            pl.BlockSpec((BATCH, EMB), lambda j: (0, 0)),
            pl.BlockSpec((_TBN, 1), lambda j: (j, 0)),
        ],
        out_specs=pl.BlockSpec((_TBN, BATCH), lambda j: (j, 0)),
        out_shape=jax.ShapeDtypeStruct((VOCAB, BATCH), jnp.float32),
        compiler_params=pltpu.CompilerParams(
            dimension_semantics=("parallel",),
        ),
    )(w_t, emb, b2)
    return out_t.T


# speedup vs baseline: 3.3358x; 1.3202x over previous
import functools

import jax
import jax.numpy as jnp
from jax import lax
from jax.experimental import pallas as pl
from jax.experimental.pallas import tpu as pltpu
from jax.experimental.pallas import tpu_sc as plsc

VOCAB = 100000
EMB = 128
BATCH = 1024

_NC, _NS = 2, 16
_NW = _NC * _NS
_B_PER_W = BATCH // _NW


def _sc_gather_body(table_hbm, idx_hbm, out_hbm, idx_v, rows_v, sem):
    wid = lax.axis_index("s") * _NC + lax.axis_index("c")
    base = wid * _B_PER_W
    pltpu.sync_copy(idx_hbm.at[pl.ds(base, _B_PER_W)], idx_v)
    pltpu.async_copy(table_hbm.at[idx_v], rows_v, sem).wait()
    pltpu.sync_copy(rows_v, out_hbm.at[pl.ds(base, _B_PER_W)])


@functools.cache
def _make_sc_gather():
    return pl.kernel(
        _sc_gather_body,
        out_type=jax.ShapeDtypeStruct((BATCH, EMB), jnp.float32),
        mesh=plsc.VectorSubcoreMesh(
            core_axis_name="c", subcore_axis_name="s",
            num_cores=_NC, num_subcores=_NS,
        ),
        scratch_types=[
            pltpu.VMEM((_B_PER_W,), jnp.int32),
            pltpu.VMEM((_B_PER_W, EMB), jnp.float32),
            pltpu.SemaphoreType.DMA,
        ],
    )


_TBN = 2048
_N_TILES = (VOCAB + _TBN - 1) // _TBN


def _mmT_body(w_ref, embt_ref, b_ref, out_ref):
    out_ref[...] = (
        jnp.dot(w_ref[...], embt_ref[...],
                preferred_element_type=jnp.float32)
        + b_ref[...].T
    )


@jax.jit
def kernel(inputs, E, W, b):
    emb = _make_sc_gather()(E, inputs.astype(jnp.int32))
    w_t = W.T
    emb_t = emb.T
    b2 = b.reshape(1, VOCAB)
    out_t = pl.pallas_call(
        _mmT_body,
        grid=(_N_TILES,),
        in_specs=[
            pl.BlockSpec((_TBN, EMB), lambda j: (j, 0)),
            pl.BlockSpec((EMB, BATCH), lambda j: (0, 0)),
            pl.BlockSpec((1, _TBN), lambda j: (0, j)),
        ],
        out_specs=pl.BlockSpec((_TBN, BATCH), lambda j: (j, 0)),
        out_shape=jax.ShapeDtypeStruct((VOCAB, BATCH), jnp.float32),
        compiler_params=pltpu.CompilerParams(
            dimension_semantics=("parallel",),
        ),
    )(w_t, emb_t, b2)
    return out_t.T


# speedup vs baseline: 3.4431x; 1.0322x over previous
import functools

import jax
import jax.numpy as jnp
from jax import lax
from jax.experimental import pallas as pl
from jax.experimental.pallas import tpu as pltpu
from jax.experimental.pallas import tpu_sc as plsc

VOCAB = 100000
EMB = 128
BATCH = 1024

_NC, _NS = 2, 16
_NW = _NC * _NS
_B_PER_W = BATCH // _NW


def _sc_gather_body(table_hbm, idx_hbm, out_hbm, idx_v, rows_v, sem):
    wid = lax.axis_index("s") * _NC + lax.axis_index("c")
    base = wid * _B_PER_W
    pltpu.sync_copy(idx_hbm.at[pl.ds(base, _B_PER_W)], idx_v)
    pltpu.async_copy(table_hbm.at[idx_v], rows_v, sem).wait()
    pltpu.sync_copy(rows_v, out_hbm.at[pl.ds(base, _B_PER_W)])


@functools.cache
def _make_sc_gather():
    return pl.kernel(
        _sc_gather_body,
        out_type=jax.ShapeDtypeStruct((BATCH, EMB), jnp.float32),
        mesh=plsc.VectorSubcoreMesh(
            core_axis_name="c", subcore_axis_name="s",
            num_cores=_NC, num_subcores=_NS,
        ),
        scratch_types=[
            pltpu.VMEM((_B_PER_W,), jnp.int32),
            pltpu.VMEM((_B_PER_W, EMB), jnp.float32),
            pltpu.SemaphoreType.DMA,
        ],
    )


_TBN = 4096
_N_TILES = (VOCAB + _TBN - 1) // _TBN


def _mmT_body(w_ref, emb_ref, b_ref, out_ref, embt_ref):
    @pl.when(pl.program_id(0) == 0)
    def _():
        embt_ref[...] = emb_ref[...].T

    out_ref[...] = (
        jnp.dot(w_ref[...], embt_ref[...],
                preferred_element_type=jnp.float32)
        + b_ref[...].T
    )


@jax.jit
def kernel(inputs, E, W, b):
    emb = _make_sc_gather()(E, inputs.astype(jnp.int32))
    w_t = W.T
    b2 = b.reshape(1, VOCAB)
    out_t = pl.pallas_call(
        _mmT_body,
        grid=(_N_TILES,),
        in_specs=[
            pl.BlockSpec((_TBN, EMB), lambda j: (j, 0)),
            pl.BlockSpec((BATCH, EMB), lambda j: (0, 0)),
            pl.BlockSpec((1, _TBN), lambda j: (0, j)),
        ],
        out_specs=pl.BlockSpec((_TBN, BATCH), lambda j: (j, 0)),
        out_shape=jax.ShapeDtypeStruct((VOCAB, BATCH), jnp.float32),
        scratch_shapes=[
            pltpu.VMEM((EMB, BATCH), jnp.float32),
        ],
        compiler_params=pltpu.CompilerParams(
            dimension_semantics=("parallel",),
        ),
    )(w_t, emb, b2)
    return out_t.T


# speedup vs baseline: 3.4627x; 1.0057x over previous
import functools

import jax
import jax.numpy as jnp
from jax import lax
from jax.experimental import pallas as pl
from jax.experimental.pallas import tpu as pltpu
from jax.experimental.pallas import tpu_sc as plsc

VOCAB = 100000
EMB = 128
BATCH = 1024

_NC, _NS = 2, 16
_NW = _NC * _NS
_B_PER_W = BATCH // _NW


def _sc_gather_body(table_hbm, idx_hbm, out_hbm, idx_v, rows_v, sem):
    wid = lax.axis_index("s") * _NC + lax.axis_index("c")
    base = wid * _B_PER_W
    pltpu.sync_copy(idx_hbm.at[pl.ds(base, _B_PER_W)], idx_v)
    pltpu.async_copy(table_hbm.at[idx_v], rows_v, sem).wait()
    pltpu.sync_copy(rows_v, out_hbm.at[pl.ds(base, _B_PER_W)])


@functools.cache
def _make_sc_gather():
    return pl.kernel(
        _sc_gather_body,
        out_type=jax.ShapeDtypeStruct((BATCH, EMB), jnp.float32),
        mesh=plsc.VectorSubcoreMesh(
            core_axis_name="c", subcore_axis_name="s",
            num_cores=_NC, num_subcores=_NS,
        ),
        scratch_types=[
            pltpu.VMEM((_B_PER_W,), jnp.int32),
            pltpu.VMEM((_B_PER_W, EMB), jnp.float32),
            pltpu.SemaphoreType.DMA,
        ],
    )


_TBN = 5120
_N_TILES = (VOCAB + _TBN - 1) // _TBN


def _mmT_body(w_ref, emb_ref, b_ref, out_ref, embt_ref):
    @pl.when(pl.program_id(0) == 0)
    def _():
        embt_ref[...] = emb_ref[...].T

    out_ref[...] = (
        jnp.dot(w_ref[...], embt_ref[...],
                preferred_element_type=jnp.float32)
        + b_ref[...].T
    )


@jax.jit
def kernel(inputs, E, W, b):
    emb = _make_sc_gather()(E, inputs.astype(jnp.int32))
    w_t = W.T
    b2 = b.reshape(1, VOCAB)
    out_t = pl.pallas_call(
        _mmT_body,
        grid=(_N_TILES,),
        in_specs=[
            pl.BlockSpec((_TBN, EMB), lambda j: (j, 0)),
            pl.BlockSpec((BATCH, EMB), lambda j: (0, 0)),
            pl.BlockSpec((1, _TBN), lambda j: (0, j)),
        ],
        out_specs=pl.BlockSpec((_TBN, BATCH), lambda j: (j, 0)),
        out_shape=jax.ShapeDtypeStruct((VOCAB, BATCH), jnp.float32),
        scratch_shapes=[
            pltpu.VMEM((EMB, BATCH), jnp.float32),
        ],
        compiler_params=pltpu.CompilerParams(
            dimension_semantics=("parallel",),
        ),
    )(w_t, emb, b2)
    return out_t.T
